# baseline (speedup 1.0000x reference)
import jax
import jax.numpy as jnp
from jax.experimental import pallas as pl
from jax.experimental.pallas import tpu as pltpu

_HI = jax.lax.Precision.HIGHEST


def _proj_kernel(x_ref, fcw_ref, fcb_ref, w0_ref, s0_ref):
    x = jnp.dot(x_ref[...], fcw_ref[...], preferred_element_type=jnp.float32,
                precision=_HI) + fcb_ref[...]
    s0 = jnp.dot(x, w0_ref[...], preferred_element_type=jnp.float32,
                 precision=_HI)
    s0_ref[...] = s0.astype(jnp.bfloat16)


def _pass_a_kernel(adj_ref, s0_ref, b0_ref, w1_ref, s1_ref):
    a = adj_ref[...].astype(jnp.bfloat16)
    t = jnp.dot(a, s0_ref[...], preferred_element_type=jnp.float32)
    h = jnp.maximum(t + b0_ref[...], 0.0)
    s1 = jnp.dot(h, w1_ref[...], preferred_element_type=jnp.float32,
                 precision=_HI)
    s1_ref[...] = s1.astype(jnp.bfloat16)


def _pass_b_kernel(adj_ref, s1_ref, b1_ref, wp_ref, bp_ref, out_ref):
    a = adj_ref[...].astype(jnp.bfloat16)
    t = jnp.dot(a, s1_ref[...], preferred_element_type=jnp.float32)
    h = jnp.maximum(t + b1_ref[...], 0.0)
    out_ref[...] = jnp.dot(h, wp_ref[...], preferred_element_type=jnp.float32,
                           precision=_HI) + bp_ref[...]


def _pick_block(n):
    for bm in (400, 500, 250, 200, 125, 100, 80, 50, 40, 25, 20, 16, 10, 8, 5, 4, 2, 1):
        if n % bm == 0:
            return bm
    return n


def kernel(X, adj, fc_W, fc_b, conv0_W, conv0_b, conv1_W, conv1_b, pred_W, pred_b):
    x0 = X[0]
    n, f_in = x0.shape
    h_dim = conv0_W.shape[1]
    out_dim = conv1_W.shape[1]
    c_dim = pred_W.shape[1]
    bm = _pick_block(n)
    grid = (n // bm,)

    fc_b2 = fc_b.reshape(1, -1)
    b0 = conv0_b.reshape(1, -1)
    b1 = conv1_b.reshape(1, -1)
    bp = pred_b.reshape(1, -1)

    whole = lambda shape: pl.BlockSpec(shape, lambda i: (0, 0))

    s0 = pl.pallas_call(
        _proj_kernel,
        out_shape=jax.ShapeDtypeStruct((n, h_dim), jnp.bfloat16),
    )(x0, fc_W, fc_b2, conv0_W)

    s1 = pl.pallas_call(
        _pass_a_kernel,
        grid=grid,
        in_specs=[
            pl.BlockSpec((bm, n), lambda i: (i, 0)),
            whole((n, h_dim)),
            whole((1, h_dim)),
            whole((h_dim, out_dim)),
        ],
        out_specs=pl.BlockSpec((bm, out_dim), lambda i: (i, 0)),
        out_shape=jax.ShapeDtypeStruct((n, out_dim), jnp.bfloat16),
    )(adj, s0, b0, conv1_W)

    out = pl.pallas_call(
        _pass_b_kernel,
        grid=grid,
        in_specs=[
            pl.BlockSpec((bm, n), lambda i: (i, 0)),
            whole((n, out_dim)),
            whole((1, out_dim)),
            whole((out_dim, c_dim)),
            whole((1, c_dim)),
        ],
        out_specs=pl.BlockSpec((bm, c_dim), lambda i: (i, 0)),
        out_shape=jax.ShapeDtypeStruct((n, c_dim), jnp.float32),
    )(adj, s1, b1, pred_W, bp)

    return out
